# Initial kernel scaffold; baseline (speedup 1.0000x reference)
import jax
import jax.numpy as jnp
from jax.experimental import pallas as pl


def kernel(x, edge_index, Wl0, bl0, Wr0, Wl1, bl1, Wr1, W_lin, b_lin):
    raise NotImplementedError("write your pallas kernel here")


# speedup vs baseline: 3.6312x; 3.6312x over previous
import functools

import jax
import jax.numpy as jnp
from jax import lax
from jax.experimental import pallas as pl
from jax.experimental.pallas import tpu as pltpu
from jax.experimental.pallas import tpu_sc as plsc

N = 10000
E = 320000
HID = 128
NC, NS, LANES = 2, 16, 16
NW = NC * NS
CH = 128
EP = ((E + (CH * NW) - 1) // (CH * NW)) * (CH * NW)
CHUNKS_PER_TILE = EP // CH // NW
N8 = ((N + 1 + NS * 8 - 1) // (NS * 8)) * (NS * 8)
ROWS_PER_TILE = N8 // NS
BLK = 1000
GRID = N // BLK

_STRIPE_CHUNKS = []
_r = 0
while _r < ROWS_PER_TILE:
    _sz = min(CH, ROWS_PER_TILE - _r)
    _STRIPE_CHUNKS.append((_r, _sz))
    _r += _sz


def _sc_mesh():
    return plsc.VectorSubcoreMesh(core_axis_name="c", subcore_axis_name="s",
                                  num_cores=NC, num_subcores=NS)


@functools.lru_cache(maxsize=None)
def _make_sum():
    def body(y_hbm, src_hbm, dst_hbm, z128_hbm, sum_out,
             sidx, didx, rows, acc):
        c = lax.axis_index("c")
        s = lax.axis_index("s")
        wid = c * NS + s
        base = s * ROWS_PER_TILE
        for r0, sz in _STRIPE_CHUNKS:
            pltpu.sync_copy(z128_hbm.at[pl.ds(r0, sz)], rows.at[pl.ds(0, sz)])
            pltpu.sync_copy(rows.at[pl.ds(0, sz)], acc.at[pl.ds(base + r0, sz)])
        plsc.subcore_barrier()

        @pl.loop(0, CHUNKS_PER_TILE)
        def _(j):
            off = (wid * CHUNKS_PER_TILE + j) * CH
            pltpu.sync_copy(src_hbm.at[pl.ds(off, CH)], sidx)
            pltpu.sync_copy(dst_hbm.at[pl.ds(off, CH)], didx)
            pltpu.sync_copy(y_hbm.at[sidx], rows)
            pltpu.sync_copy(rows, acc.at[didx], add=True)

        plsc.subcore_barrier()
        sum_c = sum_out.at[c]
        for r0, sz in _STRIPE_CHUNKS:
            pltpu.sync_copy(acc.at[pl.ds(base + r0, sz)], rows.at[pl.ds(0, sz)])
            pltpu.sync_copy(rows.at[pl.ds(0, sz)],
                            sum_c.at[pl.ds(base + r0, sz)])

    return pl.kernel(
        body,
        out_type=jax.ShapeDtypeStruct((NC, N8, HID), jnp.float32),
        mesh=_sc_mesh(),
        scratch_types=[
            pltpu.VMEM((CH,), jnp.int32),
            pltpu.VMEM((CH,), jnp.int32),
            pltpu.VMEM((CH, HID), jnp.float32),
            pltpu.VMEM_SHARED((N8, HID), jnp.float32),
        ])


@functools.lru_cache(maxsize=None)
def _make_cnt():
    def body(dst_hbm, z128_hbm, ones_hbm, cnt_out, didx, onesv, cbuf, cacc):
        c = lax.axis_index("c")
        s = lax.axis_index("s")
        wid = c * NS + s
        base = s * ROWS_PER_TILE
        for r0, sz in _STRIPE_CHUNKS:
            pltpu.sync_copy(z128_hbm.at[pl.ds(r0, sz)], cbuf.at[pl.ds(0, sz)])
            pltpu.sync_copy(cbuf.at[pl.ds(0, sz)],
                            cacc.at[pl.ds(base + r0, sz)])
        pltpu.sync_copy(ones_hbm, onesv)
        plsc.subcore_barrier()

        @pl.loop(0, CHUNKS_PER_TILE)
        def _(j):
            off = (wid * CHUNKS_PER_TILE + j) * CH
            pltpu.sync_copy(dst_hbm.at[pl.ds(off, CH)], didx)
            pltpu.sync_copy(onesv, cacc.at[didx], add=True)

        plsc.subcore_barrier()
        cnt_c = cnt_out.at[c]
        for r0, sz in _STRIPE_CHUNKS:
            pltpu.sync_copy(cacc.at[pl.ds(base + r0, sz)],
                            cbuf.at[pl.ds(0, sz)])
            pltpu.sync_copy(cbuf.at[pl.ds(0, sz)],
                            cnt_c.at[pl.ds(base + r0, sz)])

    return pl.kernel(
        body,
        out_type=jax.ShapeDtypeStruct((NC, N8, HID), jnp.float32),
        mesh=_sc_mesh(),
        scratch_types=[
            pltpu.VMEM((CH,), jnp.int32),
            pltpu.VMEM((CH, HID), jnp.float32),
            pltpu.VMEM((CH, HID), jnp.float32),
            pltpu.VMEM_SHARED((N8, HID), jnp.float32),
        ])


def _dot_t(a, w):
    return lax.dot_general(a, w, (((1,), (1,)), ((), ())),
                           precision=lax.Precision.HIGHEST,
                           preferred_element_type=jnp.float32)


def _dense0_body(x_ref, wl_ref, wr_ref, bl_ref, y0_ref, r0_ref):
    xv = x_ref[...]
    y0_ref[...] = _dot_t(xv, wl_ref[...])
    r0_ref[...] = _dot_t(xv, wr_ref[...]) + bl_ref[...]


def _dense1_body(s0a_ref, s0b_ref, ca_ref, cb_ref, r0_ref, wl_ref, wr_ref,
                 bl_ref, h1_ref, y1_ref, r1_ref, inv_ref):
    cnt = ca_ref[...][:, 0:1] + cb_ref[...][:, 0:1]
    inv = 1.0 / jnp.maximum(cnt, 1.0)
    h1 = jnp.maximum((s0a_ref[...] + s0b_ref[...]) * inv + r0_ref[...], 0.0)
    h1_ref[...] = h1
    y1_ref[...] = _dot_t(h1, wl_ref[...])
    r1_ref[...] = _dot_t(h1, wr_ref[...]) + bl_ref[...]
    inv_ref[...] = jnp.broadcast_to(inv, h1.shape)


def _dense2_body(s1a_ref, s1b_ref, inv_ref, r1_ref, h1_ref, wa_ref, wb_ref,
                 bl_ref, out_ref):
    h2 = jnp.maximum((s1a_ref[...] + s1b_ref[...]) * inv_ref[...]
                     + r1_ref[...], 0.0)
    t = _dot_t(h1_ref[...], wa_ref[...]) + _dot_t(h2, wb_ref[...]) + bl_ref[...]
    m = jnp.max(t, axis=1, keepdims=True)
    lse = jnp.log(jnp.sum(jnp.exp(t - m), axis=1, keepdims=True))
    out_ref[...] = t - m - lse


def _row_spec():
    return pl.BlockSpec((BLK, HID), lambda i: (i, 0))


def _cnt_spec():
    return pl.BlockSpec((BLK, LANES), lambda i: (i, 0))


def _full_spec(shape):
    return pl.BlockSpec(shape, lambda i: tuple(0 for _ in shape))


_dense0 = pl.pallas_call(
    _dense0_body,
    grid=(GRID,),
    in_specs=[_row_spec(), _full_spec((HID, HID)), _full_spec((HID, HID)),
              _full_spec((1, HID))],
    out_specs=[_row_spec(), _row_spec()],
    out_shape=[jax.ShapeDtypeStruct((N, HID), jnp.float32)] * 2,
)

_dense1 = pl.pallas_call(
    _dense1_body,
    grid=(GRID,),
    in_specs=[_row_spec(), _row_spec(), _row_spec(), _row_spec(), _row_spec(),
              _full_spec((HID, HID)), _full_spec((HID, HID)),
              _full_spec((1, HID))],
    out_specs=[_row_spec()] * 4,
    out_shape=[jax.ShapeDtypeStruct((N, HID), jnp.float32)] * 4,
)

_dense2 = pl.pallas_call(
    _dense2_body,
    grid=(GRID,),
    in_specs=[_row_spec(), _row_spec(), _row_spec(), _row_spec(), _row_spec(),
              _full_spec((HID, HID)), _full_spec((HID, HID)),
              _full_spec((1, HID))],
    out_specs=_row_spec(),
    out_shape=jax.ShapeDtypeStruct((N, HID), jnp.float32),
)


def kernel(x, edge_index, Wl0, bl0, Wr0, Wl1, bl1, Wr1, W_lin, b_lin):
    src = edge_index[0]
    dst = edge_index[1]
    pad = EP - E
    srcp = jnp.concatenate([src, jnp.zeros((pad,), jnp.int32)])
    dstp = jnp.concatenate([dst, jnp.full((pad,), N, jnp.int32)])
    z128 = jnp.zeros((N8, HID), jnp.float32)
    ones = jnp.ones((CH, HID), jnp.float32)

    cnt = _make_cnt()(dstp, z128, ones)
    y0, r0 = _dense0(x, Wl0, Wr0, bl0.reshape(1, HID))
    sum0 = _make_sum()(y0, srcp, dstp, z128)
    h1, y1, r1, inv = _dense1(sum0[0, :N], sum0[1, :N], cnt[0, :N],
                              cnt[1, :N], r0, Wl1, Wr1, bl1.reshape(1, HID))
    sum1 = _make_sum()(y1, srcp, dstp, z128)
    out = _dense2(sum1[0, :N], sum1[1, :N], inv, r1, h1,
                  W_lin[:, :HID], W_lin[:, HID:], b_lin.reshape(1, HID))
    return out


# speedup vs baseline: 8.3136x; 2.2895x over previous
import functools

import jax
import jax.numpy as jnp
from jax import lax
from jax.experimental import pallas as pl
from jax.experimental.pallas import tpu as pltpu
from jax.experimental.pallas import tpu_sc as plsc

N = 10000
E = 320000
HID = 128
NC, NS, LANES = 2, 16, 16
NW = NC * NS
CH = 128
CHUNKS_PER_TILE = 80
EP = CH * NW * CHUNKS_PER_TILE
N8 = ((N + 1 + NS * 8 - 1) // (NS * 8)) * (NS * 8)
ROWS_PER_TILE = N8 // NS
BLK = 1000
GRID = N // BLK

_STRIPE_CHUNKS = []
_r = 0
while _r < ROWS_PER_TILE:
    _sz = min(CH, ROWS_PER_TILE - _r)
    _STRIPE_CHUNKS.append((_r, _sz))
    _r += _sz


def _sc_mesh():
    return plsc.VectorSubcoreMesh(core_axis_name="c", subcore_axis_name="s",
                                  num_cores=NC, num_subcores=NS)


@functools.lru_cache(maxsize=None)
def _make_sum():
    NCH = CHUNKS_PER_TILE

    def body(y_hbm, src_hbm, dst_hbm, z128_hbm, sum_out,
             sidx0, sidx1, sidx2, sidx3, didx0, didx1, didx2, didx3,
             rows0, rows1, acc,
             isem0, isem1, isem2, isem3, gsem0, gsem1, ssem0, ssem1):
        sidx = [sidx0, sidx1, sidx2, sidx3]
        didx = [didx0, didx1, didx2, didx3]
        rows = [rows0, rows1]
        isem = [isem0, isem1, isem2, isem3]
        gsem = [gsem0, gsem1]
        ssem = [ssem0, ssem1]
        c = lax.axis_index("c")
        s = lax.axis_index("s")
        wid = c * NS + s
        cbase = wid * NCH
        base = s * ROWS_PER_TILE
        for r0, sz in _STRIPE_CHUNKS:
            pltpu.sync_copy(z128_hbm.at[pl.ds(r0, sz)], rows0.at[pl.ds(0, sz)])
            pltpu.sync_copy(rows0.at[pl.ds(0, sz)],
                            acc.at[pl.ds(base + r0, sz)])
        plsc.subcore_barrier()

        def idx_start(j, q):
            off = (cbase + j) * CH
            pltpu.async_copy(src_hbm.at[pl.ds(off, CH)], sidx[q], isem[q])
            pltpu.async_copy(dst_hbm.at[pl.ds(off, CH)], didx[q], isem[q])

        def idx_wait(j, q):
            off = (cbase + j) * CH
            pltpu.make_async_copy(src_hbm.at[pl.ds(off, CH)], sidx[q],
                                  isem[q]).wait()
            pltpu.make_async_copy(dst_hbm.at[pl.ds(off, CH)], didx[q],
                                  isem[q]).wait()

        def gat_start(q, b):
            pltpu.async_copy(y_hbm.at[sidx[q]], rows[b], gsem[b])

        def gat_wait(q, b):
            pltpu.make_async_copy(y_hbm.at[sidx[q]], rows[b], gsem[b]).wait()

        def sct_start(q, b):
            pltpu.async_copy(rows[b], acc.at[didx[q]], ssem[b], add=True)

        def sct_wait(q, b):
            pltpu.make_async_copy(rows[b], acc.at[didx[q]], ssem[b]).wait()

        def step(j, k, first, last):
            if not (first and k == 0):
                sct_wait((k + 3) % 4, (k + 1) % 2)
            if not (last and k == 3):
                idx_wait(j + 1, (k + 1) % 4)
                gat_start((k + 1) % 4, (k + 1) % 2)
            if not (last and k >= 1):
                idx_start(j + 3, (k + 3) % 4)
            gat_wait(k, k % 2)
            sct_start(k, k % 2)

        idx_start(0, 0)
        idx_start(1, 1)
        idx_start(2, 2)
        idx_wait(0, 0)
        gat_start(0, 0)
        for k in range(4):
            step(k, k, first=True, last=False)

        @pl.loop(1, NCH // 4 - 1)
        def _(g):
            j0 = 4 * g
            for k in range(4):
                step(j0 + k, k, first=False, last=False)

        j0 = NCH - 4
        for k in range(4):
            step(j0 + k, k, first=False, last=True)
        sct_wait(3, 1)

        plsc.subcore_barrier()
        sum_c = sum_out.at[c]
        for r0, sz in _STRIPE_CHUNKS:
            pltpu.sync_copy(acc.at[pl.ds(base + r0, sz)], rows0.at[pl.ds(0, sz)])
            pltpu.sync_copy(rows0.at[pl.ds(0, sz)],
                            sum_c.at[pl.ds(base + r0, sz)])

    return pl.kernel(
        body,
        out_type=jax.ShapeDtypeStruct((NC, N8, HID), jnp.float32),
        mesh=_sc_mesh(),
        scratch_types=(
            [pltpu.VMEM((CH,), jnp.int32)] * 8 +
            [pltpu.VMEM((CH, HID), jnp.float32)] * 2 +
            [pltpu.VMEM_SHARED((N8, HID), jnp.float32)] +
            [pltpu.SemaphoreType.DMA] * 8))


@functools.lru_cache(maxsize=None)
def _make_cnt():
    def body(dst_hbm, z128_hbm, ones_hbm, cnt_out, didx, onesv, cbuf, cacc):
        c = lax.axis_index("c")
        s = lax.axis_index("s")
        wid = c * NS + s
        base = s * ROWS_PER_TILE
        for r0, sz in _STRIPE_CHUNKS:
            pltpu.sync_copy(z128_hbm.at[pl.ds(r0, sz)], cbuf.at[pl.ds(0, sz)])
            pltpu.sync_copy(cbuf.at[pl.ds(0, sz)],
                            cacc.at[pl.ds(base + r0, sz)])
        pltpu.sync_copy(ones_hbm, onesv)
        plsc.subcore_barrier()

        @pl.loop(0, CHUNKS_PER_TILE)
        def _(j):
            off = (wid * CHUNKS_PER_TILE + j) * CH
            pltpu.sync_copy(dst_hbm.at[pl.ds(off, CH)], didx)
            pltpu.sync_copy(onesv, cacc.at[didx], add=True)

        plsc.subcore_barrier()
        cnt_c = cnt_out.at[c]
        for r0, sz in _STRIPE_CHUNKS:
            pltpu.sync_copy(cacc.at[pl.ds(base + r0, sz)],
                            cbuf.at[pl.ds(0, sz)])
            pltpu.sync_copy(cbuf.at[pl.ds(0, sz)],
                            cnt_c.at[pl.ds(base + r0, sz)])

    return pl.kernel(
        body,
        out_type=jax.ShapeDtypeStruct((NC, N8, HID), jnp.float32),
        mesh=_sc_mesh(),
        scratch_types=[
            pltpu.VMEM((CH,), jnp.int32),
            pltpu.VMEM((CH, HID), jnp.float32),
            pltpu.VMEM((CH, HID), jnp.float32),
            pltpu.VMEM_SHARED((N8, HID), jnp.float32),
        ])


def _dot_t(a, w):
    return lax.dot_general(a, w, (((1,), (1,)), ((), ())),
                           precision=lax.Precision.HIGHEST,
                           preferred_element_type=jnp.float32)


def _dense0_body(x_ref, wl_ref, wr_ref, bl_ref, y0_ref, r0_ref):
    xv = x_ref[...]
    y0_ref[...] = _dot_t(xv, wl_ref[...])
    r0_ref[...] = _dot_t(xv, wr_ref[...]) + bl_ref[...]


def _dense1_body(s0a_ref, s0b_ref, ca_ref, cb_ref, r0_ref, wl_ref, wr_ref,
                 bl_ref, h1_ref, y1_ref, r1_ref, inv_ref):
    cnt = ca_ref[...][:, 0:1] + cb_ref[...][:, 0:1]
    inv = 1.0 / jnp.maximum(cnt, 1.0)
    h1 = jnp.maximum((s0a_ref[...] + s0b_ref[...]) * inv + r0_ref[...], 0.0)
    h1_ref[...] = h1
    y1_ref[...] = _dot_t(h1, wl_ref[...])
    r1_ref[...] = _dot_t(h1, wr_ref[...]) + bl_ref[...]
    inv_ref[...] = jnp.broadcast_to(inv, h1.shape)


def _dense2_body(s1a_ref, s1b_ref, inv_ref, r1_ref, h1_ref, wa_ref, wb_ref,
                 bl_ref, out_ref):
    h2 = jnp.maximum((s1a_ref[...] + s1b_ref[...]) * inv_ref[...]
                     + r1_ref[...], 0.0)
    t = _dot_t(h1_ref[...], wa_ref[...]) + _dot_t(h2, wb_ref[...]) + bl_ref[...]
    m = jnp.max(t, axis=1, keepdims=True)
    lse = jnp.log(jnp.sum(jnp.exp(t - m), axis=1, keepdims=True))
    out_ref[...] = t - m - lse


def _row_spec():
    return pl.BlockSpec((BLK, HID), lambda i: (i, 0))


def _cnt_spec():
    return pl.BlockSpec((BLK, LANES), lambda i: (i, 0))


def _full_spec(shape):
    return pl.BlockSpec(shape, lambda i: tuple(0 for _ in shape))


_dense0 = pl.pallas_call(
    _dense0_body,
    grid=(GRID,),
    in_specs=[_row_spec(), _full_spec((HID, HID)), _full_spec((HID, HID)),
              _full_spec((1, HID))],
    out_specs=[_row_spec(), _row_spec()],
    out_shape=[jax.ShapeDtypeStruct((N, HID), jnp.float32)] * 2,
)

_dense1 = pl.pallas_call(
    _dense1_body,
    grid=(GRID,),
    in_specs=[_row_spec(), _row_spec(), _row_spec(), _row_spec(), _row_spec(),
              _full_spec((HID, HID)), _full_spec((HID, HID)),
              _full_spec((1, HID))],
    out_specs=[_row_spec()] * 4,
    out_shape=[jax.ShapeDtypeStruct((N, HID), jnp.float32)] * 4,
)

_dense2 = pl.pallas_call(
    _dense2_body,
    grid=(GRID,),
    in_specs=[_row_spec(), _row_spec(), _row_spec(), _row_spec(), _row_spec(),
              _full_spec((HID, HID)), _full_spec((HID, HID)),
              _full_spec((1, HID))],
    out_specs=_row_spec(),
    out_shape=jax.ShapeDtypeStruct((N, HID), jnp.float32),
)


def kernel(x, edge_index, Wl0, bl0, Wr0, Wl1, bl1, Wr1, W_lin, b_lin):
    src = edge_index[0]
    dst = edge_index[1]
    pad = EP - E
    pad_ids = jnp.arange(pad, dtype=jnp.int32)
    srcp = jnp.concatenate([src, pad_ids % N])
    dstp = jnp.concatenate([dst, N + pad_ids % (N8 - N)])
    z128 = jnp.zeros((N8, HID), jnp.float32)
    ones = jnp.ones((CH, HID), jnp.float32)

    cnt = _make_cnt()(dstp, z128, ones)
    y0, r0 = _dense0(x, Wl0, Wr0, bl0.reshape(1, HID))
    sum0 = _make_sum()(y0, srcp, dstp, z128)
    h1, y1, r1, inv = _dense1(sum0[0, :N], sum0[1, :N], cnt[0, :N],
                              cnt[1, :N], r0, Wl1, Wr1, bl1.reshape(1, HID))
    sum1 = _make_sum()(y1, srcp, dstp, z128)
    out = _dense2(sum1[0, :N], sum1[1, :N], inv, r1, h1,
                  W_lin[:, :HID], W_lin[:, HID:], b_lin.reshape(1, HID))
    return out


# speedup vs baseline: 10.8284x; 1.3025x over previous
import dataclasses
import functools

import jax
import jax.numpy as jnp
from jax import lax
from jax.experimental import pallas as pl
from jax.experimental.pallas import tpu as pltpu
from jax.experimental.pallas import tpu_sc as plsc

N = 10000
E = 320000
HID = 128
NC, NS, LANES = 2, 16, 16
NW = NC * NS
CH = 128
CHUNKS_PER_TILE = 80
EP = CH * NW * CHUNKS_PER_TILE
N8 = ((N + 1 + NS * 8 - 1) // (NS * 8)) * (NS * 8)
ROWS_PER_TILE = N8 // NS
BLK = 1000
GRID = N // BLK

_STRIPE_CHUNKS = []
_r = 0
while _r < ROWS_PER_TILE:
    _sz = min(CH, ROWS_PER_TILE - _r)
    _STRIPE_CHUNKS.append((_r, _sz))
    _r += _sz


def _sc_mesh():
    return plsc.VectorSubcoreMesh(core_axis_name="c", subcore_axis_name="s",
                                  num_cores=NC, num_subcores=NS)


def _sc_params():
    cp = pltpu.CompilerParams()
    if "needs_layout_passes" in pltpu.CompilerParams.__dataclass_fields__:
        cp = dataclasses.replace(cp, needs_layout_passes=False)
    return cp


@functools.lru_cache(maxsize=None)
def _make_sum(with_cnt):
    NCH = CHUNKS_PER_TILE

    def body(*refs):
        if with_cnt:
            (y_hbm, src_hbm, dst_hbm, z128_hbm, znode_hbm, sum_out, cnt_out,
             sidx0, sidx1, sidx2, sidx3, didx0, didx1, didx2, didx3,
             rows0, rows1, acc, cloc,
             isem0, isem1, isem2, isem3, gsem0, gsem1, ssem0, ssem1) = refs
        else:
            (y_hbm, src_hbm, dst_hbm, z128_hbm, sum_out,
             sidx0, sidx1, sidx2, sidx3, didx0, didx1, didx2, didx3,
             rows0, rows1, acc,
             isem0, isem1, isem2, isem3, gsem0, gsem1, ssem0, ssem1) = refs
        sidx = [sidx0, sidx1, sidx2, sidx3]
        didx = [didx0, didx1, didx2, didx3]
        rows = [rows0, rows1]
        isem = [isem0, isem1, isem2, isem3]
        gsem = [gsem0, gsem1]
        ssem = [ssem0, ssem1]
        c = lax.axis_index("c")
        s = lax.axis_index("s")
        wid = c * NS + s
        cbase = wid * NCH
        base = s * ROWS_PER_TILE
        for r0, sz in _STRIPE_CHUNKS:
            pltpu.sync_copy(z128_hbm.at[pl.ds(r0, sz)], rows0.at[pl.ds(0, sz)])
            pltpu.sync_copy(rows0.at[pl.ds(0, sz)],
                            acc.at[pl.ds(base + r0, sz)])
        if with_cnt:
            pltpu.sync_copy(znode_hbm, cloc)
        plsc.subcore_barrier()

        def idx_start(j, q):
            off = (cbase + j) * CH
            pltpu.async_copy(src_hbm.at[pl.ds(off, CH)], sidx[q], isem[q])
            pltpu.async_copy(dst_hbm.at[pl.ds(off, CH)], didx[q], isem[q])

        def idx_wait(j, q):
            off = (cbase + j) * CH
            pltpu.make_async_copy(src_hbm.at[pl.ds(off, CH)], sidx[q],
                                  isem[q]).wait()
            pltpu.make_async_copy(dst_hbm.at[pl.ds(off, CH)], didx[q],
                                  isem[q]).wait()

        def gat_start(q, b):
            pltpu.async_copy(y_hbm.at[sidx[q]], rows[b], gsem[b])

        def gat_wait(q, b):
            pltpu.make_async_copy(y_hbm.at[sidx[q]], rows[b], gsem[b]).wait()

        def sct_start(q, b):
            pltpu.async_copy(rows[b], acc.at[didx[q]], ssem[b], add=True)

        def sct_wait(q, b):
            pltpu.make_async_copy(rows[b], acc.at[didx[q]], ssem[b]).wait()

        def step(j, k, first, last):
            if not (first and k == 0):
                sct_wait((k + 3) % 4, (k + 1) % 2)
            if not (last and k == 3):
                idx_wait(j + 1, (k + 1) % 4)
                gat_start((k + 1) % 4, (k + 1) % 2)
            if not (last and k >= 1):
                idx_start(j + 3, (k + 3) % 4)
            gat_wait(k, k % 2)
            sct_start(k, k % 2)
            if with_cnt:
                ones16 = jnp.full((16,), 1.0, jnp.float32)
                for k16 in range(CH // 16):
                    idx16 = didx[k][pl.ds(16 * k16, 16)]
                    plsc.addupdate_scatter(cloc, [idx16], ones16)

        idx_start(0, 0)
        idx_start(1, 1)
        idx_start(2, 2)
        idx_wait(0, 0)
        gat_start(0, 0)
        for k in range(4):
            step(k, k, first=True, last=False)

        @pl.loop(1, NCH // 4 - 1)
        def _(g):
            j0 = 4 * g
            for k in range(4):
                step(j0 + k, k, first=False, last=False)

        j0 = NCH - 4
        for k in range(4):
            step(j0 + k, k, first=False, last=True)
        sct_wait(3, 1)

        plsc.subcore_barrier()
        sum_c = sum_out.at[c]
        for r0, sz in _STRIPE_CHUNKS:
            pltpu.sync_copy(acc.at[pl.ds(base + r0, sz)], rows0.at[pl.ds(0, sz)])
            pltpu.sync_copy(rows0.at[pl.ds(0, sz)],
                            sum_c.at[pl.ds(base + r0, sz)])
        if with_cnt:
            pltpu.sync_copy(cloc, cnt_out.at[wid])

    out_type = [jax.ShapeDtypeStruct((NC, N8, HID), jnp.float32)]
    scratch = (
        [pltpu.VMEM((CH,), jnp.int32)] * 8 +
        [pltpu.VMEM((CH, HID), jnp.float32)] * 2 +
        [pltpu.VMEM_SHARED((N8, HID), jnp.float32)])
    if with_cnt:
        out_type.append(jax.ShapeDtypeStruct((NW, N8), jnp.float32))
        scratch += [pltpu.VMEM((N8,), jnp.float32)]
    scratch += [pltpu.SemaphoreType.DMA] * 8

    return pl.kernel(
        body,
        out_type=out_type,
        mesh=_sc_mesh(),
        compiler_params=_sc_params(),
        scratch_types=scratch)


def _dot_t(a, w):
    return lax.dot_general(a, w, (((1,), (1,)), ((), ())),
                           precision=lax.Precision.HIGHEST,
                           preferred_element_type=jnp.float32)


def _dense0_body(x_ref, wl_ref, wr_ref, bl_ref, y0_ref, r0_ref):
    xv = x_ref[...]
    y0_ref[...] = _dot_t(xv, wl_ref[...])
    r0_ref[...] = _dot_t(xv, wr_ref[...]) + bl_ref[...]


def _dense1_body(s0a_ref, s0b_ref, cnt_ref, r0_ref, wl_ref, wr_ref,
                 bl_ref, h1_ref, y1_ref, r1_ref, inv_ref):
    cnt = lax.dot_general(cnt_ref[0], jnp.ones((NW, 1), jnp.float32),
                          (((0,), (0,)), ((), ())),
                          precision=lax.Precision.HIGHEST,
                          preferred_element_type=jnp.float32)
    inv = 1.0 / jnp.maximum(cnt, 1.0)
    h1 = jnp.maximum((s0a_ref[...] + s0b_ref[...]) * inv + r0_ref[...], 0.0)
    h1_ref[...] = h1
    y1_ref[...] = _dot_t(h1, wl_ref[...])
    r1_ref[...] = _dot_t(h1, wr_ref[...]) + bl_ref[...]
    inv_ref[...] = jnp.broadcast_to(inv, h1.shape)


def _dense2_body(s1a_ref, s1b_ref, inv_ref, r1_ref, h1_ref, wa_ref, wb_ref,
                 bl_ref, out_ref):
    h2 = jnp.maximum((s1a_ref[...] + s1b_ref[...]) * inv_ref[...]
                     + r1_ref[...], 0.0)
    t = _dot_t(h1_ref[...], wa_ref[...]) + _dot_t(h2, wb_ref[...]) + bl_ref[...]
    m = jnp.max(t, axis=1, keepdims=True)
    lse = jnp.log(jnp.sum(jnp.exp(t - m), axis=1, keepdims=True))
    out_ref[...] = t - m - lse


def _row_spec():
    return pl.BlockSpec((BLK, HID), lambda i: (i, 0))


def _cnt_spec():
    return pl.BlockSpec((BLK, LANES), lambda i: (i, 0))


def _full_spec(shape):
    return pl.BlockSpec(shape, lambda i: tuple(0 for _ in shape))


_dense0 = pl.pallas_call(
    _dense0_body,
    grid=(GRID,),
    in_specs=[_row_spec(), _full_spec((HID, HID)), _full_spec((HID, HID)),
              _full_spec((1, HID))],
    out_specs=[_row_spec(), _row_spec()],
    out_shape=[jax.ShapeDtypeStruct((N, HID), jnp.float32)] * 2,
)

_dense1 = pl.pallas_call(
    _dense1_body,
    grid=(GRID,),
    in_specs=[_row_spec(), _row_spec(),
              pl.BlockSpec((1, NW, BLK), lambda i: (i, 0, 0)), _row_spec(),
              _full_spec((HID, HID)), _full_spec((HID, HID)),
              _full_spec((1, HID))],
    out_specs=[_row_spec()] * 4,
    out_shape=[jax.ShapeDtypeStruct((N, HID), jnp.float32)] * 4,
)

_dense2 = pl.pallas_call(
    _dense2_body,
    grid=(GRID,),
    in_specs=[_row_spec(), _row_spec(), _row_spec(), _row_spec(), _row_spec(),
              _full_spec((HID, HID)), _full_spec((HID, HID)),
              _full_spec((1, HID))],
    out_specs=_row_spec(),
    out_shape=jax.ShapeDtypeStruct((N, HID), jnp.float32),
)


def kernel(x, edge_index, Wl0, bl0, Wr0, Wl1, bl1, Wr1, W_lin, b_lin):
    src = edge_index[0]
    dst = edge_index[1]
    pad = EP - E
    pad_ids = jnp.arange(pad, dtype=jnp.int32)
    srcp = jnp.concatenate([src, pad_ids % N])
    dstp = jnp.concatenate([dst, N + pad_ids % (N8 - N)])
    z128 = jnp.zeros((N8, HID), jnp.float32)
    znode = jnp.zeros((N8,), jnp.float32)

    y0, r0 = _dense0(x, Wl0, Wr0, bl0.reshape(1, HID))
    sum0, cnt = _make_sum(True)(y0, srcp, dstp, z128, znode)
    cntr = cnt[:, :N].reshape(NW, GRID, BLK).transpose(1, 0, 2)
    h1, y1, r1, inv = _dense1(sum0[0, :N], sum0[1, :N], cntr,
                              r0, Wl1, Wr1, bl1.reshape(1, HID))
    (sum1,) = _make_sum(False)(y1, srcp, dstp, z128)
    out = _dense2(sum1[0, :N], sum1[1, :N], inv, r1, h1,
                  W_lin[:, :HID], W_lin[:, HID:], b_lin.reshape(1, HID))
    return out


# speedup vs baseline: 12.0567x; 1.1134x over previous
import dataclasses
import functools

import jax
import jax.numpy as jnp
import numpy as np
from jax import lax
from jax.experimental import pallas as pl
from jax.experimental.pallas import tpu as pltpu
from jax.experimental.pallas import tpu_sc as plsc

N = 10000
E = 320000
HID = 128
NC, NS, LANES = 2, 16, 16
NW = NC * NS
CH = 128
CHUNKS_PER_TILE = 80
EP = CH * NW * CHUNKS_PER_TILE
N8 = ((N + 1 + NS * 8 - 1) // (NS * 8)) * (NS * 8)
ROWS_PER_TILE = N8 // NS
BLK = 2000
GRID = N // BLK

_STRIPE_CHUNKS = []
_r = 0
while _r < ROWS_PER_TILE:
    _sz = min(CH, ROWS_PER_TILE - _r)
    _STRIPE_CHUNKS.append((_r, _sz))
    _r += _sz


def _sc_mesh():
    return plsc.VectorSubcoreMesh(core_axis_name="c", subcore_axis_name="s",
                                  num_cores=NC, num_subcores=NS)


def _sc_params():
    cp = pltpu.CompilerParams()
    if "needs_layout_passes" in pltpu.CompilerParams.__dataclass_fields__:
        cp = dataclasses.replace(cp, needs_layout_passes=False)
    return cp


@functools.lru_cache(maxsize=None)
def _make_sum(with_cnt):
    NCH = CHUNKS_PER_TILE

    def body(*refs):
        if with_cnt:
            (y_hbm, src_hbm, dst_hbm, z128_hbm, znode_hbm, sum_out, cnt_out,
             sidx0, sidx1, sidx2, sidx3, didx0, didx1, didx2, didx3,
             rows0, rows1, acc, cloc,
             isem0, isem1, isem2, isem3, gsem0, gsem1, ssem0, ssem1) = refs
        else:
            (y_hbm, src_hbm, dst_hbm, z128_hbm, sum_out,
             sidx0, sidx1, sidx2, sidx3, didx0, didx1, didx2, didx3,
             rows0, rows1, acc,
             isem0, isem1, isem2, isem3, gsem0, gsem1, ssem0, ssem1) = refs
        sidx = [sidx0, sidx1, sidx2, sidx3]
        didx = [didx0, didx1, didx2, didx3]
        rows = [rows0, rows1]
        isem = [isem0, isem1, isem2, isem3]
        gsem = [gsem0, gsem1]
        ssem = [ssem0, ssem1]
        c = lax.axis_index("c")
        s = lax.axis_index("s")
        wid = c * NS + s
        cbase = wid * NCH
        base = s * ROWS_PER_TILE
        for r0, sz in _STRIPE_CHUNKS:
            pltpu.sync_copy(z128_hbm.at[pl.ds(r0, sz)], rows0.at[pl.ds(0, sz)])
            pltpu.sync_copy(rows0.at[pl.ds(0, sz)],
                            acc.at[pl.ds(base + r0, sz)])
        if with_cnt:
            pltpu.sync_copy(znode_hbm, cloc)
        plsc.subcore_barrier()

        def idx_start(j, q):
            off = (cbase + j) * CH
            pltpu.async_copy(src_hbm.at[pl.ds(off, CH)], sidx[q], isem[q])
            pltpu.async_copy(dst_hbm.at[pl.ds(off, CH)], didx[q], isem[q])

        def idx_wait(j, q):
            off = (cbase + j) * CH
            pltpu.make_async_copy(src_hbm.at[pl.ds(off, CH)], sidx[q],
                                  isem[q]).wait()
            pltpu.make_async_copy(dst_hbm.at[pl.ds(off, CH)], didx[q],
                                  isem[q]).wait()

        def gat_start(q, b):
            pltpu.async_copy(y_hbm.at[sidx[q]], rows[b], gsem[b])

        def gat_wait(q, b):
            pltpu.make_async_copy(y_hbm.at[sidx[q]], rows[b], gsem[b]).wait()

        def sct_start(q, b):
            pltpu.async_copy(rows[b], acc.at[didx[q]], ssem[b], add=True)

        def sct_wait(q, b):
            pltpu.make_async_copy(rows[b], acc.at[didx[q]], ssem[b]).wait()

        def step(j, k, first, last):
            if not (first and k == 0):
                sct_wait((k + 3) % 4, (k + 1) % 2)
            if not (last and k == 3):
                idx_wait(j + 1, (k + 1) % 4)
                gat_start((k + 1) % 4, (k + 1) % 2)
            if not (last and k >= 1):
                idx_start(j + 3, (k + 3) % 4)
            gat_wait(k, k % 2)
            sct_start(k, k % 2)
            if with_cnt:
                ones16 = jnp.full((16,), 1.0, jnp.float32)
                for k16 in range(CH // 16):
                    idx16 = didx[k][pl.ds(16 * k16, 16)]
                    plsc.addupdate_scatter(cloc, [idx16], ones16)

        idx_start(0, 0)
        idx_start(1, 1)
        idx_start(2, 2)
        idx_wait(0, 0)
        gat_start(0, 0)
        for k in range(4):
            step(k, k, first=True, last=False)

        @pl.loop(1, NCH // 4 - 1)
        def _(g):
            j0 = 4 * g
            for k in range(4):
                step(j0 + k, k, first=False, last=False)

        j0 = NCH - 4
        for k in range(4):
            step(j0 + k, k, first=False, last=True)
        sct_wait(3, 1)

        plsc.subcore_barrier()
        sum_c = sum_out.at[c]
        for r0, sz in _STRIPE_CHUNKS:
            pltpu.sync_copy(acc.at[pl.ds(base + r0, sz)], rows0.at[pl.ds(0, sz)])
            pltpu.sync_copy(rows0.at[pl.ds(0, sz)],
                            sum_c.at[pl.ds(base + r0, sz)])
        if with_cnt:
            pltpu.sync_copy(cloc, cnt_out.at[wid])

    out_type = [jax.ShapeDtypeStruct((NC, N8, HID), jnp.float32)]
    scratch = (
        [pltpu.VMEM((CH,), jnp.int32)] * 8 +
        [pltpu.VMEM((CH, HID), jnp.float32)] * 2 +
        [pltpu.VMEM_SHARED((N8, HID), jnp.float32)])
    if with_cnt:
        out_type.append(jax.ShapeDtypeStruct((NW, N8), jnp.float32))
        scratch += [pltpu.VMEM((N8,), jnp.float32)]
    scratch += [pltpu.SemaphoreType.DMA] * 8

    return pl.kernel(
        body,
        out_type=out_type,
        mesh=_sc_mesh(),
        compiler_params=_sc_params(),
        scratch_types=scratch)


def _dot_t(a, w):
    return lax.dot_general(a, w, (((1,), (1,)), ((), ())),
                           precision=lax.Precision.HIGHEST,
                           preferred_element_type=jnp.float32)


def _d_y0_body(x_ref, wl_ref, y0_ref):
    y0_ref[...] = _dot_t(x_ref[...], wl_ref[...])


def _d_r0_body(x_ref, wr_ref, bl_ref, r0_ref):
    r0_ref[...] = _dot_t(x_ref[...], wr_ref[...]) + bl_ref[...]


def _d_h1_body(s0a_ref, s0b_ref, cnt_ref, r0_ref, wl_ref,
               h1_ref, y1_ref, inv_ref):
    cnt = lax.dot_general(cnt_ref[0], jnp.ones((NW, 1), jnp.float32),
                          (((0,), (0,)), ((), ())),
                          precision=lax.Precision.HIGHEST,
                          preferred_element_type=jnp.float32)
    inv = 1.0 / jnp.maximum(cnt, 1.0)
    h1 = jnp.maximum((s0a_ref[0] + s0b_ref[0]) * inv + r0_ref[...], 0.0)
    h1_ref[...] = h1
    y1_ref[...] = _dot_t(h1, wl_ref[...])
    inv_ref[...] = jnp.broadcast_to(inv, h1.shape)


def _d_t1_body(h1_ref, wr_ref, bl_ref, wa_ref, bo_ref, r1_ref, t1_ref):
    h1 = h1_ref[...]
    r1_ref[...] = _dot_t(h1, wr_ref[...]) + bl_ref[...]
    t1_ref[...] = _dot_t(h1, wa_ref[...]) + bo_ref[...]


def _d_out_body(s1a_ref, s1b_ref, inv_ref, r1_ref, t1_ref, wb_ref, out_ref):
    h2 = jnp.maximum((s1a_ref[0] + s1b_ref[0]) * inv_ref[...]
                     + r1_ref[...], 0.0)
    t = t1_ref[...] + _dot_t(h2, wb_ref[...])
    m = jnp.max(t, axis=1, keepdims=True)
    lse = jnp.log(jnp.sum(jnp.exp(t - m), axis=1, keepdims=True))
    out_ref[...] = t - m - lse


def _row_spec():
    return pl.BlockSpec((BLK, HID), lambda i: (i, 0))


def _sum_spec(slab):
    return pl.BlockSpec((1, BLK, HID), lambda i, _a=slab: (_a, i, 0))


def _full_spec(shape):
    return pl.BlockSpec(shape, lambda i: tuple(0 for _ in shape))


_ROW_OUT = jax.ShapeDtypeStruct((N, HID), jnp.float32)

_d_y0 = pl.pallas_call(
    _d_y0_body, grid=(GRID,),
    in_specs=[_row_spec(), _full_spec((HID, HID))],
    out_specs=_row_spec(), out_shape=_ROW_OUT,
)

_d_r0 = pl.pallas_call(
    _d_r0_body, grid=(GRID,),
    in_specs=[_row_spec(), _full_spec((HID, HID)), _full_spec((1, HID))],
    out_specs=_row_spec(), out_shape=_ROW_OUT,
)

_d_h1 = pl.pallas_call(
    _d_h1_body, grid=(GRID,),
    in_specs=[_sum_spec(0), _sum_spec(1),
              pl.BlockSpec((1, NW, BLK), lambda i: (i, 0, 0)), _row_spec(),
              _full_spec((HID, HID))],
    out_specs=[_row_spec()] * 3,
    out_shape=[_ROW_OUT] * 3,
)

_d_t1 = pl.pallas_call(
    _d_t1_body, grid=(GRID,),
    in_specs=[_row_spec(), _full_spec((HID, HID)), _full_spec((1, HID)),
              _full_spec((HID, HID)), _full_spec((1, HID))],
    out_specs=[_row_spec()] * 2,
    out_shape=[_ROW_OUT] * 2,
)

_d_out = pl.pallas_call(
    _d_out_body, grid=(GRID,),
    in_specs=[_sum_spec(0), _sum_spec(1), _row_spec(), _row_spec(),
              _row_spec(), _full_spec((HID, HID))],
    out_specs=_row_spec(), out_shape=_ROW_OUT,
)

_PAD_SRC = np.arange(EP - E, dtype=np.int32) % N
_PAD_DST = N + np.arange(EP - E, dtype=np.int32) % (N8 - N)


def kernel(x, edge_index, Wl0, bl0, Wr0, Wl1, bl1, Wr1, W_lin, b_lin):
    srcp = jnp.concatenate([edge_index[0], jnp.asarray(_PAD_SRC)])
    dstp = jnp.concatenate([edge_index[1], jnp.asarray(_PAD_DST)])
    z128 = jnp.zeros((N8, HID), jnp.float32)
    znode = jnp.zeros((N8,), jnp.float32)

    y0 = _d_y0(x, Wl0)
    r0 = _d_r0(x, Wr0, bl0.reshape(1, HID))
    sum0, cnt = _make_sum(True)(y0, srcp, dstp, z128, znode)
    cntr = cnt[:, :N].reshape(NW, GRID, BLK).transpose(1, 0, 2)
    h1, y1, inv = _d_h1(sum0, sum0, cntr, r0, Wl1)
    r1, t1 = _d_t1(h1, Wr1, bl1.reshape(1, HID),
                   W_lin[:, :HID], b_lin.reshape(1, HID))
    (sum1,) = _make_sum(False)(y1, srcp, dstp, z128)
    return _d_out(sum1, sum1, inv, r1, t1, W_lin[:, HID:])


# speedup vs baseline: 13.3437x; 1.1067x over previous
import dataclasses
import functools

import jax
import jax.numpy as jnp
import numpy as np
from jax import lax
from jax.experimental import pallas as pl
from jax.experimental.pallas import tpu as pltpu
from jax.experimental.pallas import tpu_sc as plsc

N = 10000
E = 320000
HID = 128
NC, NS, LANES = 2, 16, 16
NW = NC * NS
CH = 128
CHUNKS_PER_TILE = 80
EP = CH * NW * CHUNKS_PER_TILE
N8 = ((N + 1 + NS * 8 - 1) // (NS * 8)) * (NS * 8)
ROWS_PER_TILE = N8 // NS
BLK = 2000
GRID = N // BLK

_STRIPE_CHUNKS = []
_r = 0
while _r < ROWS_PER_TILE:
    _sz = min(CH, ROWS_PER_TILE - _r)
    _STRIPE_CHUNKS.append((_r, _sz))
    _r += _sz


def _sc_mesh():
    return plsc.VectorSubcoreMesh(core_axis_name="c", subcore_axis_name="s",
                                  num_cores=NC, num_subcores=NS)


def _sc_params():
    cp = pltpu.CompilerParams()
    if "needs_layout_passes" in pltpu.CompilerParams.__dataclass_fields__:
        cp = dataclasses.replace(cp, needs_layout_passes=False)
    return cp


@functools.lru_cache(maxsize=None)
def _make_sum(with_cnt):
    NCH = CHUNKS_PER_TILE

    def body(*refs):
        if with_cnt:
            (y_hbm, sd_hbm, z128_hbm, znode_hbm, sum_out, cnt_out,
             sidx0, sidx1, sidx2, sidx3, didx0, didx1, didx2, didx3,
             rows0, rows1, acc, cloc,
             isem0, isem1, isem2, isem3, gsem0, gsem1, ssem0, ssem1) = refs
        else:
            (y_hbm, sd_hbm, z128_hbm, sum_out,
             sidx0, sidx1, sidx2, sidx3, didx0, didx1, didx2, didx3,
             rows0, rows1, acc,
             isem0, isem1, isem2, isem3, gsem0, gsem1, ssem0, ssem1) = refs
        sidx = [sidx0, sidx1, sidx2, sidx3]
        didx = [didx0, didx1, didx2, didx3]
        rows = [rows0, rows1]
        isem = [isem0, isem1, isem2, isem3]
        gsem = [gsem0, gsem1]
        ssem = [ssem0, ssem1]
        c = lax.axis_index("c")
        s = lax.axis_index("s")
        wid = c * NS + s
        cbase = wid * NCH
        base = s * ROWS_PER_TILE
        pltpu.sync_copy(z128_hbm.at[pl.ds(0, CH)], rows0)
        for r0, sz in _STRIPE_CHUNKS:
            pltpu.async_copy(rows0.at[pl.ds(0, sz)],
                             acc.at[pl.ds(base + r0, sz)], gsem0)
        if with_cnt:
            pltpu.sync_copy(znode_hbm, cloc)
        for r0, sz in _STRIPE_CHUNKS:
            pltpu.make_async_copy(rows0.at[pl.ds(0, sz)],
                                  acc.at[pl.ds(base + r0, sz)], gsem0).wait()
        plsc.subcore_barrier()

        def idx_start(j, q):
            cj = cbase + j
            pltpu.async_copy(sd_hbm.at[0, cj], sidx[q], isem[q])
            pltpu.async_copy(sd_hbm.at[1, cj], didx[q], isem[q])

        def idx_wait(j, q):
            cj = cbase + j
            pltpu.make_async_copy(sd_hbm.at[0, cj], sidx[q], isem[q]).wait()
            pltpu.make_async_copy(sd_hbm.at[1, cj], didx[q], isem[q]).wait()

        def gat_start(q, b):
            pltpu.async_copy(y_hbm.at[sidx[q]], rows[b], gsem[b])

        def gat_wait(q, b):
            pltpu.make_async_copy(y_hbm.at[sidx[q]], rows[b], gsem[b]).wait()

        def sct_start(q, b):
            pltpu.async_copy(rows[b], acc.at[didx[q]], ssem[b], add=True)

        def sct_wait(q, b):
            pltpu.make_async_copy(rows[b], acc.at[didx[q]], ssem[b]).wait()

        def step(j, k, first, last):
            if not (first and k == 0):
                sct_wait((k + 3) % 4, (k + 1) % 2)
            if not (last and k == 3):
                idx_wait(j + 1, (k + 1) % 4)
                gat_start((k + 1) % 4, (k + 1) % 2)
            if not (last and k >= 1):
                idx_start(j + 3, (k + 3) % 4)
            gat_wait(k, k % 2)
            sct_start(k, k % 2)
            if with_cnt:
                ones16 = jnp.full((16,), 1.0, jnp.float32)
                for k16 in range(CH // 16):
                    idx16 = didx[k][pl.ds(16 * k16, 16)]
                    plsc.addupdate_scatter(cloc, [idx16], ones16)

        idx_start(0, 0)
        idx_start(1, 1)
        idx_start(2, 2)
        idx_wait(0, 0)
        gat_start(0, 0)
        for k in range(4):
            step(k, k, first=True, last=False)

        @pl.loop(1, NCH // 4 - 1)
        def _(g):
            j0 = 4 * g
            for k in range(4):
                step(j0 + k, k, first=False, last=False)

        j0 = NCH - 4
        for k in range(4):
            step(j0 + k, k, first=False, last=True)
        sct_wait(3, 1)

        plsc.subcore_barrier()
        sum_c = sum_out.at[c]
        nd = len(_STRIPE_CHUNKS)
        dbuf = [rows0, rows1]

        def d1_start(ci):
            r0, sz = _STRIPE_CHUNKS[ci]
            pltpu.async_copy(acc.at[pl.ds(base + r0, sz)],
                             dbuf[ci % 2].at[pl.ds(0, sz)], gsem[ci % 2])

        def d1_wait(ci):
            r0, sz = _STRIPE_CHUNKS[ci]
            pltpu.make_async_copy(acc.at[pl.ds(base + r0, sz)],
                                  dbuf[ci % 2].at[pl.ds(0, sz)],
                                  gsem[ci % 2]).wait()

        def d2_start(ci):
            r0, sz = _STRIPE_CHUNKS[ci]
            pltpu.async_copy(dbuf[ci % 2].at[pl.ds(0, sz)],
                             sum_c.at[pl.ds(base + r0, sz)], ssem[ci % 2])

        def d2_wait(ci):
            r0, sz = _STRIPE_CHUNKS[ci]
            pltpu.make_async_copy(dbuf[ci % 2].at[pl.ds(0, sz)],
                                  sum_c.at[pl.ds(base + r0, sz)],
                                  ssem[ci % 2]).wait()

        d1_start(0)
        d1_start(1)
        if with_cnt:
            pltpu.async_copy(cloc, cnt_out.at[wid], isem0)
        for ci in range(nd):
            d1_wait(ci)
            d2_start(ci)
            if ci + 2 < nd:
                d2_wait(ci)
                d1_start(ci + 2)
        d2_wait(nd - 2)
        d2_wait(nd - 1)
        if with_cnt:
            pltpu.make_async_copy(cloc, cnt_out.at[wid], isem0).wait()

    out_type = [jax.ShapeDtypeStruct((NC, N8, HID), jnp.float32)]
    scratch = (
        [pltpu.VMEM((CH,), jnp.int32)] * 8 +
        [pltpu.VMEM((CH, HID), jnp.float32)] * 2 +
        [pltpu.VMEM_SHARED((N8, HID), jnp.float32)])
    if with_cnt:
        out_type.append(jax.ShapeDtypeStruct((NW, N8), jnp.float32))
        scratch += [pltpu.VMEM((N8,), jnp.float32)]
    scratch += [pltpu.SemaphoreType.DMA] * 8

    return pl.kernel(
        body,
        out_type=out_type,
        mesh=_sc_mesh(),
        compiler_params=_sc_params(),
        scratch_types=scratch)


def _dot_t(a, w):
    return lax.dot_general(a, w, (((1,), (1,)), ((), ())),
                           precision=lax.Precision.HIGHEST,
                           preferred_element_type=jnp.float32)


def _d_y0_body(x_ref, wl_ref, y0_ref):
    y0_ref[...] = _dot_t(x_ref[...], wl_ref[...])


def _d_r0_body(x_ref, wr_ref, bl_ref, r0_ref):
    r0_ref[...] = _dot_t(x_ref[...], wr_ref[...]) + bl_ref[...]


def _d_h1_body(s0a_ref, s0b_ref, cnt_ref, r0_ref, wl_ref,
               h1_ref, y1_ref, inv_ref):
    cnt = lax.dot_general(cnt_ref[0], jnp.ones((NW, 1), jnp.float32),
                          (((0,), (0,)), ((), ())),
                          precision=lax.Precision.HIGHEST,
                          preferred_element_type=jnp.float32)
    inv = 1.0 / jnp.maximum(cnt, 1.0)
    h1 = jnp.maximum((s0a_ref[0] + s0b_ref[0]) * inv + r0_ref[...], 0.0)
    h1_ref[...] = h1
    y1_ref[...] = _dot_t(h1, wl_ref[...])
    inv_ref[...] = jnp.broadcast_to(inv, h1.shape)


def _d_t1_body(h1_ref, wr_ref, bl_ref, wa_ref, bo_ref, r1_ref, t1_ref):
    h1 = h1_ref[...]
    r1_ref[...] = _dot_t(h1, wr_ref[...]) + bl_ref[...]
    t1_ref[...] = _dot_t(h1, wa_ref[...]) + bo_ref[...]


def _d_out_body(s1a_ref, s1b_ref, inv_ref, r1_ref, t1_ref, wb_ref, out_ref):
    h2 = jnp.maximum((s1a_ref[0] + s1b_ref[0]) * inv_ref[...]
                     + r1_ref[...], 0.0)
    t = t1_ref[...] + _dot_t(h2, wb_ref[...])
    m = jnp.max(t, axis=1, keepdims=True)
    lse = jnp.log(jnp.sum(jnp.exp(t - m), axis=1, keepdims=True))
    out_ref[...] = t - m - lse


def _row_spec():
    return pl.BlockSpec((BLK, HID), lambda i: (i, 0))


def _sum_spec(slab):
    return pl.BlockSpec((1, BLK, HID), lambda i, _a=slab: (_a, i, 0))


def _full_spec(shape):
    return pl.BlockSpec(shape, lambda i: tuple(0 for _ in shape))


_ROW_OUT = jax.ShapeDtypeStruct((N, HID), jnp.float32)

_d_y0 = pl.pallas_call(
    _d_y0_body, grid=(GRID,),
    in_specs=[_row_spec(), _full_spec((HID, HID))],
    out_specs=_row_spec(), out_shape=_ROW_OUT,
)

_d_r0 = pl.pallas_call(
    _d_r0_body, grid=(GRID,),
    in_specs=[_row_spec(), _full_spec((HID, HID)), _full_spec((1, HID))],
    out_specs=_row_spec(), out_shape=_ROW_OUT,
)

_d_h1 = pl.pallas_call(
    _d_h1_body, grid=(GRID,),
    in_specs=[_sum_spec(0), _sum_spec(1),
              pl.BlockSpec((1, NW, BLK), lambda i: (i, 0, 0)), _row_spec(),
              _full_spec((HID, HID))],
    out_specs=[_row_spec()] * 3,
    out_shape=[_ROW_OUT] * 3,
)

_d_t1 = pl.pallas_call(
    _d_t1_body, grid=(GRID,),
    in_specs=[_row_spec(), _full_spec((HID, HID)), _full_spec((1, HID)),
              _full_spec((HID, HID)), _full_spec((1, HID))],
    out_specs=[_row_spec()] * 2,
    out_shape=[_ROW_OUT] * 2,
)

_d_out = pl.pallas_call(
    _d_out_body, grid=(GRID,),
    in_specs=[_sum_spec(0), _sum_spec(1), _row_spec(), _row_spec(),
              _row_spec(), _full_spec((HID, HID))],
    out_specs=_row_spec(), out_shape=_ROW_OUT,
)

_NCHUNKS = EP // CH
_PAD_SD = np.stack([np.arange(EP - E, dtype=np.int32) % N,
                    N + np.arange(EP - E, dtype=np.int32) % (N8 - N)]
                   ).reshape(2, (EP - E) // CH, CH)


def kernel(x, edge_index, Wl0, bl0, Wr0, Wl1, bl1, Wr1, W_lin, b_lin):
    sd = jnp.concatenate([edge_index.reshape(2, E // CH, CH),
                          jnp.asarray(_PAD_SD)], axis=1)
    z128 = jnp.zeros((N8, HID), jnp.float32)
    znode = jnp.zeros((N8,), jnp.float32)

    y0 = _d_y0(x, Wl0)
    r0 = _d_r0(x, Wr0, bl0.reshape(1, HID))
    sum0, cnt = _make_sum(True)(y0, sd, z128, znode)
    cntr = cnt[:, :N].reshape(NW, GRID, BLK).transpose(1, 0, 2)
    h1, y1, inv = _d_h1(sum0, sum0, cntr, r0, Wl1)
    r1, t1 = _d_t1(h1, Wr1, bl1.reshape(1, HID),
                   W_lin[:, :HID], b_lin.reshape(1, HID))
    (sum1,) = _make_sum(False)(y1, sd, z128)
    return _d_out(sum1, sum1, inv, r1, t1, W_lin[:, HID:])


# speedup vs baseline: 14.7226x; 1.1033x over previous
import dataclasses
import functools

import jax
import jax.numpy as jnp
import numpy as np
from jax import lax
from jax.experimental import pallas as pl
from jax.experimental.pallas import tpu as pltpu
from jax.experimental.pallas import tpu_sc as plsc

N = 10000
E = 320000
HID = 128
NC, NS, LANES = 2, 16, 16
NW = NC * NS
CH = 64
CHUNKS_PER_TILE = 160
EP = CH * NW * CHUNKS_PER_TILE
N8 = ((N + 1 + NS * 8 - 1) // (NS * 8)) * (NS * 8)
ROWS_PER_TILE = N8 // NS
BLK = 2000
GRID = N // BLK

_STRIPE_CHUNKS = []
_r = 0
while _r < ROWS_PER_TILE:
    _sz = min(CH, ROWS_PER_TILE - _r)
    _STRIPE_CHUNKS.append((_r, _sz))
    _r += _sz


def _sc_mesh():
    return plsc.VectorSubcoreMesh(core_axis_name="c", subcore_axis_name="s",
                                  num_cores=NC, num_subcores=NS)


def _sc_params():
    cp = pltpu.CompilerParams()
    if "needs_layout_passes" in pltpu.CompilerParams.__dataclass_fields__:
        cp = dataclasses.replace(cp, needs_layout_passes=False)
    return cp


@functools.lru_cache(maxsize=None)
def _make_sum(with_cnt):
    NCH = CHUNKS_PER_TILE

    def body(*refs):
        if with_cnt:
            (y_hbm, sd_hbm, z128_hbm, znode_hbm, sum_out, cnt_out,
             *bufs) = refs
        else:
            (y_hbm, sd_hbm, z128_hbm, sum_out, *bufs) = refs
        sidx = bufs[0:8]
        didx = bufs[8:16]
        rows = bufs[16:20]
        if with_cnt:
            acc, cloc = bufs[20], bufs[21]
            sems = bufs[22:]
        else:
            acc = bufs[20]
            sems = bufs[21:]
        isem = sems[0:8]
        gsem = sems[8:12]
        ssem = sems[12:16]
        c = lax.axis_index("c")
        s = lax.axis_index("s")
        wid = c * NS + s
        cbase = wid * NCH
        base = s * ROWS_PER_TILE
        pltpu.sync_copy(z128_hbm.at[pl.ds(0, CH)], rows[0])
        for r0, sz in _STRIPE_CHUNKS:
            pltpu.async_copy(rows[0].at[pl.ds(0, sz)],
                             acc.at[pl.ds(base + r0, sz)], gsem[0])
        if with_cnt:
            pltpu.sync_copy(znode_hbm, cloc)
        for r0, sz in _STRIPE_CHUNKS:
            pltpu.make_async_copy(rows[0].at[pl.ds(0, sz)],
                                  acc.at[pl.ds(base + r0, sz)], gsem[0]).wait()
        plsc.subcore_barrier()

        def idx_start(j, q):
            cj = cbase + j
            pltpu.async_copy(sd_hbm.at[0, cj], sidx[q], isem[q])
            pltpu.async_copy(sd_hbm.at[1, cj], didx[q], isem[q])

        def idx_wait(j, q):
            cj = cbase + j
            pltpu.make_async_copy(sd_hbm.at[0, cj], sidx[q], isem[q]).wait()
            pltpu.make_async_copy(sd_hbm.at[1, cj], didx[q], isem[q]).wait()

        def gat_start(q, b):
            pltpu.async_copy(y_hbm.at[sidx[q]], rows[b], gsem[b])

        def gat_wait(q, b):
            pltpu.make_async_copy(y_hbm.at[sidx[q]], rows[b], gsem[b]).wait()

        def sct_start(q, b):
            pltpu.async_copy(rows[b], acc.at[didx[q]], ssem[b], add=True)

        def sct_wait(q, b):
            pltpu.make_async_copy(rows[b], acc.at[didx[q]], ssem[b]).wait()

        def step(j, k, first, last):
            if not (first and k == 0):
                sct_wait((k + 7) % 8, (k + 3) % 4)
            if not (last and k >= 5):
                idx_wait(j + 3, (k + 3) % 8)
                gat_start((k + 3) % 8, (k + 3) % 4)
            if not (last and k >= 3):
                idx_start(j + 5, (k + 5) % 8)
            gat_wait(k, k % 4)
            sct_start(k, k % 4)
            if with_cnt:
                ones16 = jnp.full((16,), 1.0, jnp.float32)
                for k16 in range(CH // 16):
                    idx16 = didx[k][pl.ds(16 * k16, 16)]
                    plsc.addupdate_scatter(cloc, [idx16], ones16)

        for q in range(5):
            idx_start(q, q)
        for q in range(3):
            idx_wait(q, q)
            gat_start(q, q)
        for k in range(8):
            step(k, k, first=True, last=False)

        @pl.loop(1, NCH // 8 - 1)
        def _(g):
            j0 = 8 * g
            for k in range(8):
                step(j0 + k, k, first=False, last=False)

        j0 = NCH - 8
        for k in range(8):
            step(j0 + k, k, first=False, last=True)
        sct_wait(7, 3)

        plsc.subcore_barrier()
        sum_c = sum_out.at[c]
        nd = len(_STRIPE_CHUNKS)
        dbuf = [rows[0], rows[1]]

        def d1_start(ci):
            r0, sz = _STRIPE_CHUNKS[ci]
            pltpu.async_copy(acc.at[pl.ds(base + r0, sz)],
                             dbuf[ci % 2].at[pl.ds(0, sz)], gsem[ci % 2])

        def d1_wait(ci):
            r0, sz = _STRIPE_CHUNKS[ci]
            pltpu.make_async_copy(acc.at[pl.ds(base + r0, sz)],
                                  dbuf[ci % 2].at[pl.ds(0, sz)],
                                  gsem[ci % 2]).wait()

        def d2_start(ci):
            r0, sz = _STRIPE_CHUNKS[ci]
            pltpu.async_copy(dbuf[ci % 2].at[pl.ds(0, sz)],
                             sum_c.at[pl.ds(base + r0, sz)], ssem[ci % 2])

        def d2_wait(ci):
            r0, sz = _STRIPE_CHUNKS[ci]
            pltpu.make_async_copy(dbuf[ci % 2].at[pl.ds(0, sz)],
                                  sum_c.at[pl.ds(base + r0, sz)],
                                  ssem[ci % 2]).wait()

        d1_start(0)
        d1_start(1)
        if with_cnt:
            pltpu.async_copy(cloc, cnt_out.at[wid], isem[0])
        for ci in range(nd):
            d1_wait(ci)
            d2_start(ci)
            if ci + 2 < nd:
                d2_wait(ci)
                d1_start(ci + 2)
        d2_wait(nd - 2)
        d2_wait(nd - 1)
        if with_cnt:
            pltpu.make_async_copy(cloc, cnt_out.at[wid], isem[0]).wait()

    out_type = [jax.ShapeDtypeStruct((NC, N8, HID), jnp.float32)]
    scratch = (
        [pltpu.VMEM((CH,), jnp.int32)] * 16 +
        [pltpu.VMEM((CH, HID), jnp.float32)] * 4 +
        [pltpu.VMEM_SHARED((N8, HID), jnp.float32)])
    if with_cnt:
        out_type.append(jax.ShapeDtypeStruct((NW, N8), jnp.float32))
        scratch += [pltpu.VMEM((N8,), jnp.float32)]
    scratch += [pltpu.SemaphoreType.DMA] * 16

    return pl.kernel(
        body,
        out_type=out_type,
        mesh=_sc_mesh(),
        compiler_params=_sc_params(),
        scratch_types=scratch)


def _dot_t(a, w):
    return lax.dot_general(a, w, (((1,), (1,)), ((), ())),
                           precision=lax.Precision.HIGHEST,
                           preferred_element_type=jnp.float32)


def _d_y0_body(x_ref, wl_ref, y0_ref):
    y0_ref[...] = _dot_t(x_ref[...], wl_ref[...])


def _d_r0_body(x_ref, wr_ref, bl_ref, r0_ref):
    r0_ref[...] = _dot_t(x_ref[...], wr_ref[...]) + bl_ref[...]


def _d_h1_body(s0a_ref, s0b_ref, cnt_ref, r0_ref, wl_ref,
               h1_ref, y1_ref, inv_ref):
    cnt = lax.dot_general(cnt_ref[0], jnp.ones((NW, 1), jnp.float32),
                          (((0,), (0,)), ((), ())),
                          precision=lax.Precision.HIGHEST,
                          preferred_element_type=jnp.float32)
    inv = 1.0 / jnp.maximum(cnt, 1.0)
    h1 = jnp.maximum((s0a_ref[0] + s0b_ref[0]) * inv + r0_ref[...], 0.0)
    h1_ref[...] = h1
    y1_ref[...] = _dot_t(h1, wl_ref[...])
    inv_ref[...] = jnp.broadcast_to(inv, h1.shape)


def _d_t1_body(h1_ref, wr_ref, bl_ref, wa_ref, bo_ref, r1_ref, t1_ref):
    h1 = h1_ref[...]
    r1_ref[...] = _dot_t(h1, wr_ref[...]) + bl_ref[...]
    t1_ref[...] = _dot_t(h1, wa_ref[...]) + bo_ref[...]


def _d_out_body(s1a_ref, s1b_ref, inv_ref, r1_ref, t1_ref, wb_ref, out_ref):
    h2 = jnp.maximum((s1a_ref[0] + s1b_ref[0]) * inv_ref[...]
                     + r1_ref[...], 0.0)
    t = t1_ref[...] + _dot_t(h2, wb_ref[...])
    m = jnp.max(t, axis=1, keepdims=True)
    lse = jnp.log(jnp.sum(jnp.exp(t - m), axis=1, keepdims=True))
    out_ref[...] = t - m - lse


def _row_spec():
    return pl.BlockSpec((BLK, HID), lambda i: (i, 0))


def _sum_spec(slab):
    return pl.BlockSpec((1, BLK, HID), lambda i, _a=slab: (_a, i, 0))


def _full_spec(shape):
    return pl.BlockSpec(shape, lambda i: tuple(0 for _ in shape))


_ROW_OUT = jax.ShapeDtypeStruct((N, HID), jnp.float32)

_d_y0 = pl.pallas_call(
    _d_y0_body, grid=(GRID,),
    in_specs=[_row_spec(), _full_spec((HID, HID))],
    out_specs=_row_spec(), out_shape=_ROW_OUT,
)

_d_r0 = pl.pallas_call(
    _d_r0_body, grid=(GRID,),
    in_specs=[_row_spec(), _full_spec((HID, HID)), _full_spec((1, HID))],
    out_specs=_row_spec(), out_shape=_ROW_OUT,
)

_d_h1 = pl.pallas_call(
    _d_h1_body, grid=(GRID,),
    in_specs=[_sum_spec(0), _sum_spec(1),
              pl.BlockSpec((1, NW, BLK), lambda i: (i, 0, 0)), _row_spec(),
              _full_spec((HID, HID))],
    out_specs=[_row_spec()] * 3,
    out_shape=[_ROW_OUT] * 3,
)

_d_t1 = pl.pallas_call(
    _d_t1_body, grid=(GRID,),
    in_specs=[_row_spec(), _full_spec((HID, HID)), _full_spec((1, HID)),
              _full_spec((HID, HID)), _full_spec((1, HID))],
    out_specs=[_row_spec()] * 2,
    out_shape=[_ROW_OUT] * 2,
)

_d_out = pl.pallas_call(
    _d_out_body, grid=(GRID,),
    in_specs=[_sum_spec(0), _sum_spec(1), _row_spec(), _row_spec(),
              _row_spec(), _full_spec((HID, HID))],
    out_specs=_row_spec(), out_shape=_ROW_OUT,
)

_NCHUNKS = EP // CH
_PAD_SD = np.stack([np.arange(EP - E, dtype=np.int32) % N,
                    N + np.arange(EP - E, dtype=np.int32) % (N8 - N)]
                   ).reshape(2, (EP - E) // CH, CH)


def kernel(x, edge_index, Wl0, bl0, Wr0, Wl1, bl1, Wr1, W_lin, b_lin):
    sd = jnp.concatenate([edge_index.reshape(2, E // CH, CH),
                          jnp.asarray(_PAD_SD)], axis=1)
    z128 = jnp.zeros((N8, HID), jnp.float32)
    znode = jnp.zeros((N8,), jnp.float32)

    y0 = _d_y0(x, Wl0)
    r0 = _d_r0(x, Wr0, bl0.reshape(1, HID))
    sum0, cnt = _make_sum(True)(y0, sd, z128, znode)
    cntr = cnt[:, :N].reshape(NW, GRID, BLK).transpose(1, 0, 2)
    h1, y1, inv = _d_h1(sum0, sum0, cntr, r0, Wl1)
    r1, t1 = _d_t1(h1, Wr1, bl1.reshape(1, HID),
                   W_lin[:, :HID], b_lin.reshape(1, HID))
    (sum1,) = _make_sum(False)(y1, sd, z128)
    return _d_out(sum1, sum1, inv, r1, t1, W_lin[:, HID:])
